# baseline (speedup 1.0000x reference)
import functools

import jax
import jax.numpy as jnp
from jax import lax
from jax.experimental import pallas as pl
from jax.experimental.pallas import tpu as pltpu
from jax.experimental.pallas import tpu_sc as plsc

EMBED = 1024
TEXT_END = 32000
IMG_OFFSET = 32000
L = 16
CH = 48
NB = 2


def _project_codebook(codebook, w):
    vq_vocab, vq_embed = codebook.shape
    bm = 512

    def body(cb_ref, w_ref, o_ref):
        o_ref[...] = lax.dot_general(
            cb_ref[...], w_ref[...],
            dimension_numbers=(((1,), (1,)), ((), ())),
            preferred_element_type=jnp.float32)

    return pl.pallas_call(
        body,
        grid=(vq_vocab // bm,),
        in_specs=[
            pl.BlockSpec((bm, vq_embed), lambda i: (i, 0)),
            pl.BlockSpec((EMBED, vq_embed), lambda i: (0, 0)),
        ],
        out_specs=pl.BlockSpec((bm, EMBED), lambda i: (i, 0)),
        out_shape=jax.ShapeDtypeStruct((vq_vocab, EMBED), jnp.float32),
    )(codebook, w)


@functools.cache
def _sc_lookup(n_tokens):
    info = plsc.get_sparse_core_info()
    nw = info.num_cores * info.num_subcores
    tpw = n_tokens // nw
    assert n_tokens % nw == 0 and tpw % L == 0
    nb = NB
    mesh = plsc.VectorSubcoreMesh(core_axis_name="c", subcore_axis_name="s")

    @functools.partial(
        pl.kernel,
        mesh=mesh,
        out_type=jax.ShapeDtypeStruct((n_tokens, EMBED), jnp.float32),
        compiler_params=pltpu.CompilerParams(needs_layout_passes=False),
        scratch_types=[
            pltpu.VMEM((tpw,), jnp.int32),
            pltpu.VMEM((tpw,), jnp.int32),
            pltpu.VMEM((tpw,), jnp.int32),
            pltpu.VMEM((tpw,), jnp.int32),
            pltpu.VMEM((tpw,), jnp.int32),
        ] + [pltpu.VMEM((CH, EMBED), jnp.float32)] * nb + [
            pltpu.SemaphoreType.DMA,
            pltpu.SemaphoreType.DMA,
        ],
    )
    def k(x_hbm, te_hbm, pc_hbm, out_hbm,
          x_v, tidx, tpos, iidx, ipos, *rest):
        bufs = rest[:nb]
        sem_g, sem_s = rest[nb], rest[nb + 1]
        wid = lax.axis_index("s") * info.num_cores + lax.axis_index("c")
        base = wid * tpw
        pltpu.sync_copy(x_hbm.at[pl.ds(base, tpw)], x_v)

        lanes = lax.iota(jnp.int32, L)

        def compact(j, carry):
            nt, ni = carry
            xv = x_v[pl.ds(j * L, L)]
            m_text = xv < TEXT_END
            m_img = jnp.logical_not(m_text)
            mt32 = m_text.astype(jnp.int32)
            incl = plsc.cumsum(mt32)
            excl = incl - mt32
            pos = base + j * L + lanes
            slot_t = nt + excl
            slot_i = ni + (lanes - excl)
            plsc.store_scatter(tidx, [slot_t], xv, mask=m_text)
            plsc.store_scatter(tpos, [slot_t], pos, mask=m_text)
            plsc.store_scatter(iidx, [slot_i], xv - IMG_OFFSET, mask=m_img)
            plsc.store_scatter(ipos, [slot_i], pos, mask=m_img)
            cnt = incl[L - 1]
            return nt + cnt, ni + (L - cnt)

        nt, ni = lax.fori_loop(0, tpw // L, compact,
                               (jnp.int32(0), jnp.int32(0)))

        zeros16 = jnp.zeros((L,), jnp.int32)

        def pad(idx_ref, pos_ref, n):
            idx0 = plsc.load_gather(idx_ref, [zeros16])
            pos0 = plsc.load_gather(pos_ref, [zeros16])
            n8 = (n + 7) & -8
            pad_end = jnp.where(n8 < CH, jnp.int32(CH), n8)
            for kk in range(CH // L):
                slot = n + kk * L + lanes
                m = slot < pad_end
                plsc.store_scatter(idx_ref, [slot], idx0, mask=m)
                plsc.store_scatter(pos_ref, [slot], pos0, mask=m)
            return jnp.where(n > 0, pad_end, jnp.int32(0))

        ne_t = pad(tidx, tpos, nt)
        ne_i = pad(iidx, ipos, ni)

        nch_t = (ne_t + CH - 1) // CH
        nch_i = (ne_i + CH - 1) // CH
        total = nch_t + nch_i
        last_t = jnp.maximum(ne_t - CH, 0)
        last_i = jnp.maximum(ne_i - CH, 0)

        def start_gather(c, b):
            @pl.when(c < nch_t)
            def _():
                s = pl.multiple_of(jnp.minimum(c * CH, last_t), 8)
                pltpu.make_async_copy(
                    te_hbm.at[tidx.at[pl.ds(s, CH)]], b, sem_g).start()

            @pl.when(c >= nch_t)
            def _():
                s = pl.multiple_of(jnp.minimum((c - nch_t) * CH, last_i), 8)
                pltpu.make_async_copy(
                    pc_hbm.at[iidx.at[pl.ds(s, CH)]], b, sem_g).start()

        def wait_gather(b):
            pltpu.make_async_copy(
                te_hbm.at[tidx.at[pl.ds(0, CH)]], b, sem_g).wait()

        def start_scatter(c, b):
            @pl.when(c < nch_t)
            def _():
                s = pl.multiple_of(jnp.minimum(c * CH, last_t), 8)
                pltpu.make_async_copy(
                    b, out_hbm.at[tpos.at[pl.ds(s, CH)]], sem_s).start()

            @pl.when(c >= nch_t)
            def _():
                s = pl.multiple_of(jnp.minimum((c - nch_t) * CH, last_i), 8)
                pltpu.make_async_copy(
                    b, out_hbm.at[ipos.at[pl.ds(s, CH)]], sem_s).start()

        def wait_scatter(b):
            pltpu.make_async_copy(
                b, out_hbm.at[tpos.at[pl.ds(0, CH)]], sem_s).wait()

        for b in range(nb):
            @pl.when(b < total)
            def _(b=b):
                start_gather(b, bufs[b])

        def group(p, c):
            g0 = p * nb
            for b in range(nb):
                j = g0 + b

                @pl.when(j < total)
                def _(j=j, b=b):
                    wait_gather(bufs[b])
                    start_scatter(j, bufs[b])

                    @pl.when(j + nb < total)
                    def _():
                        wait_scatter(bufs[b])
                        start_gather(j + nb, bufs[b])
            return c

        lax.fori_loop(0, (total + nb - 1) // nb, group, 0)
        for b in range(nb):
            @pl.when(b < total)
            def _(b=b):
                wait_scatter(bufs[b])

    return k


def kernel(x, token_embedding, vqgan_codebook, vqgan_proj_W):
    pc = _project_codebook(vqgan_codebook, vqgan_proj_W)
    n_tokens = x.shape[0] * x.shape[1]
    out = _sc_lookup(n_tokens)(x.reshape(-1), token_embedding, pc)
    return out.reshape(x.shape + (EMBED,))


# speedup vs baseline: 1.0164x; 1.0164x over previous
import functools

import jax
import jax.numpy as jnp
from jax import lax
from jax.experimental import pallas as pl
from jax.experimental.pallas import tpu as pltpu
from jax.experimental.pallas import tpu_sc as plsc

EMBED = 1024
TEXT_END = 32000
IMG_OFFSET = 32000
L = 16
CH = 40
NB = 3


def _project_codebook(codebook, w):
    vq_vocab, vq_embed = codebook.shape
    bm = 512

    def body(cb_ref, w_ref, o_ref):
        o_ref[...] = lax.dot_general(
            cb_ref[...], w_ref[...],
            dimension_numbers=(((1,), (1,)), ((), ())),
            preferred_element_type=jnp.float32)

    return pl.pallas_call(
        body,
        grid=(vq_vocab // bm,),
        in_specs=[
            pl.BlockSpec((bm, vq_embed), lambda i: (i, 0)),
            pl.BlockSpec((EMBED, vq_embed), lambda i: (0, 0)),
        ],
        out_specs=pl.BlockSpec((bm, EMBED), lambda i: (i, 0)),
        out_shape=jax.ShapeDtypeStruct((vq_vocab, EMBED), jnp.float32),
    )(codebook, w)


@functools.cache
def _sc_lookup(n_tokens):
    info = plsc.get_sparse_core_info()
    nw = info.num_cores * info.num_subcores
    tpw = n_tokens // nw
    assert n_tokens % nw == 0 and tpw % L == 0
    nb = NB
    mesh = plsc.VectorSubcoreMesh(core_axis_name="c", subcore_axis_name="s")

    @functools.partial(
        pl.kernel,
        mesh=mesh,
        out_type=jax.ShapeDtypeStruct((n_tokens, EMBED), jnp.float32),
        compiler_params=pltpu.CompilerParams(needs_layout_passes=False),
        scratch_types=[
            pltpu.VMEM((tpw,), jnp.int32),
            pltpu.VMEM((tpw,), jnp.int32),
            pltpu.VMEM((tpw,), jnp.int32),
            pltpu.VMEM((tpw,), jnp.int32),
            pltpu.VMEM((tpw,), jnp.int32),
        ] + [pltpu.VMEM((CH, EMBED), jnp.float32)] * nb + [
            pltpu.SemaphoreType.DMA,
            pltpu.SemaphoreType.DMA,
        ],
    )
    def k(x_hbm, te_hbm, pc_hbm, out_hbm,
          x_v, tidx, tpos, iidx, ipos, *rest):
        bufs = rest[:nb]
        sem_g, sem_s = rest[nb], rest[nb + 1]
        wid = lax.axis_index("s") * info.num_cores + lax.axis_index("c")
        base = wid * tpw
        pltpu.sync_copy(x_hbm.at[pl.ds(base, tpw)], x_v)

        lanes = lax.iota(jnp.int32, L)

        def compact(j, carry):
            nt, ni = carry
            xv = x_v[pl.ds(j * L, L)]
            m_text = xv < TEXT_END
            m_img = jnp.logical_not(m_text)
            mt32 = m_text.astype(jnp.int32)
            incl = plsc.cumsum(mt32)
            excl = incl - mt32
            pos = base + j * L + lanes
            slot_t = nt + excl
            slot_i = ni + (lanes - excl)
            plsc.store_scatter(tidx, [slot_t], xv, mask=m_text)
            plsc.store_scatter(tpos, [slot_t], pos, mask=m_text)
            plsc.store_scatter(iidx, [slot_i], xv - IMG_OFFSET, mask=m_img)
            plsc.store_scatter(ipos, [slot_i], pos, mask=m_img)
            cnt = incl[L - 1]
            return nt + cnt, ni + (L - cnt)

        nt, ni = lax.fori_loop(0, tpw // L, compact,
                               (jnp.int32(0), jnp.int32(0)))

        zeros16 = jnp.zeros((L,), jnp.int32)

        def pad(idx_ref, pos_ref, n):
            idx0 = plsc.load_gather(idx_ref, [zeros16])
            pos0 = plsc.load_gather(pos_ref, [zeros16])
            n8 = (n + 7) & -8
            pad_end = jnp.where(n8 < CH, jnp.int32(CH), n8)
            for kk in range(CH // L):
                slot = n + kk * L + lanes
                m = slot < pad_end
                plsc.store_scatter(idx_ref, [slot], idx0, mask=m)
                plsc.store_scatter(pos_ref, [slot], pos0, mask=m)
            return jnp.where(n > 0, pad_end, jnp.int32(0))

        ne_t = pad(tidx, tpos, nt)
        ne_i = pad(iidx, ipos, ni)

        nch_t = (ne_t + CH - 1) // CH
        nch_i = (ne_i + CH - 1) // CH
        total = nch_t + nch_i
        last_t = jnp.maximum(ne_t - CH, 0)
        last_i = jnp.maximum(ne_i - CH, 0)

        def start_gather(c, b):
            @pl.when(c < nch_t)
            def _():
                s = pl.multiple_of(jnp.minimum(c * CH, last_t), 8)
                pltpu.make_async_copy(
                    te_hbm.at[tidx.at[pl.ds(s, CH)]], b, sem_g).start()

            @pl.when(c >= nch_t)
            def _():
                s = pl.multiple_of(jnp.minimum((c - nch_t) * CH, last_i), 8)
                pltpu.make_async_copy(
                    pc_hbm.at[iidx.at[pl.ds(s, CH)]], b, sem_g).start()

        def wait_gather(b):
            pltpu.make_async_copy(
                te_hbm.at[tidx.at[pl.ds(0, CH)]], b, sem_g).wait()

        def start_scatter(c, b):
            @pl.when(c < nch_t)
            def _():
                s = pl.multiple_of(jnp.minimum(c * CH, last_t), 8)
                pltpu.make_async_copy(
                    b, out_hbm.at[tpos.at[pl.ds(s, CH)]], sem_s).start()

            @pl.when(c >= nch_t)
            def _():
                s = pl.multiple_of(jnp.minimum((c - nch_t) * CH, last_i), 8)
                pltpu.make_async_copy(
                    b, out_hbm.at[ipos.at[pl.ds(s, CH)]], sem_s).start()

        def wait_scatter(b):
            pltpu.make_async_copy(
                b, out_hbm.at[tpos.at[pl.ds(0, CH)]], sem_s).wait()

        for b in range(nb):
            @pl.when(b < total)
            def _(b=b):
                start_gather(b, bufs[b])

        def group(p, c):
            g0 = p * nb
            for b in range(nb):
                j = g0 + b

                @pl.when(j < total)
                def _(j=j, b=b):
                    wait_gather(bufs[b])
                    start_scatter(j, bufs[b])

                    @pl.when(j + nb < total)
                    def _():
                        wait_scatter(bufs[b])
                        start_gather(j + nb, bufs[b])
            return c

        lax.fori_loop(0, (total + nb - 1) // nb, group, 0)
        for b in range(nb):
            @pl.when(b < total)
            def _(b=b):
                wait_scatter(bufs[b])

    return k


def kernel(x, token_embedding, vqgan_codebook, vqgan_proj_W):
    pc = _project_codebook(vqgan_codebook, vqgan_proj_W)
    n_tokens = x.shape[0] * x.shape[1]
    out = _sc_lookup(n_tokens)(x.reshape(-1), token_embedding, pc)
    return out.reshape(x.shape + (EMBED,))


# speedup vs baseline: 1.0351x; 1.0184x over previous
import functools

import jax
import jax.numpy as jnp
from jax import lax
from jax.experimental import pallas as pl
from jax.experimental.pallas import tpu as pltpu
from jax.experimental.pallas import tpu_sc as plsc

EMBED = 1024
TEXT_END = 32000
IMG_OFFSET = 32000
L = 16
CH = 24
NB = 4


def _project_codebook(codebook, w):
    vq_vocab, vq_embed = codebook.shape
    bm = 512

    def body(cb_ref, w_ref, o_ref):
        o_ref[...] = lax.dot_general(
            cb_ref[...], w_ref[...],
            dimension_numbers=(((1,), (1,)), ((), ())),
            preferred_element_type=jnp.float32)

    return pl.pallas_call(
        body,
        grid=(vq_vocab // bm,),
        in_specs=[
            pl.BlockSpec((bm, vq_embed), lambda i: (i, 0)),
            pl.BlockSpec((EMBED, vq_embed), lambda i: (0, 0)),
        ],
        out_specs=pl.BlockSpec((bm, EMBED), lambda i: (i, 0)),
        out_shape=jax.ShapeDtypeStruct((vq_vocab, EMBED), jnp.float32),
    )(codebook, w)


@functools.cache
def _sc_lookup(n_tokens):
    info = plsc.get_sparse_core_info()
    nw = info.num_cores * info.num_subcores
    tpw = n_tokens // nw
    assert n_tokens % nw == 0 and tpw % L == 0
    nb = NB
    mesh = plsc.VectorSubcoreMesh(core_axis_name="c", subcore_axis_name="s")

    @functools.partial(
        pl.kernel,
        mesh=mesh,
        out_type=jax.ShapeDtypeStruct((n_tokens, EMBED), jnp.float32),
        compiler_params=pltpu.CompilerParams(needs_layout_passes=False),
        scratch_types=[
            pltpu.VMEM((tpw,), jnp.int32),
            pltpu.VMEM((tpw,), jnp.int32),
            pltpu.VMEM((tpw,), jnp.int32),
            pltpu.VMEM((tpw,), jnp.int32),
            pltpu.VMEM((tpw,), jnp.int32),
        ] + [pltpu.VMEM((CH, EMBED), jnp.float32)] * nb + [
            pltpu.SemaphoreType.DMA,
            pltpu.SemaphoreType.DMA,
        ],
    )
    def k(x_hbm, te_hbm, pc_hbm, out_hbm,
          x_v, tidx, tpos, iidx, ipos, *rest):
        bufs = rest[:nb]
        sem_g, sem_s = rest[nb], rest[nb + 1]
        wid = lax.axis_index("s") * info.num_cores + lax.axis_index("c")
        base = wid * tpw
        pltpu.sync_copy(x_hbm.at[pl.ds(base, tpw)], x_v)

        lanes = lax.iota(jnp.int32, L)

        def compact(j, carry):
            nt, ni = carry
            xv = x_v[pl.ds(j * L, L)]
            m_text = xv < TEXT_END
            m_img = jnp.logical_not(m_text)
            mt32 = m_text.astype(jnp.int32)
            incl = plsc.cumsum(mt32)
            excl = incl - mt32
            pos = base + j * L + lanes
            slot_t = nt + excl
            slot_i = ni + (lanes - excl)
            plsc.store_scatter(tidx, [slot_t], xv, mask=m_text)
            plsc.store_scatter(tpos, [slot_t], pos, mask=m_text)
            plsc.store_scatter(iidx, [slot_i], xv - IMG_OFFSET, mask=m_img)
            plsc.store_scatter(ipos, [slot_i], pos, mask=m_img)
            cnt = incl[L - 1]
            return nt + cnt, ni + (L - cnt)

        nt, ni = lax.fori_loop(0, tpw // L, compact,
                               (jnp.int32(0), jnp.int32(0)))

        zeros16 = jnp.zeros((L,), jnp.int32)

        def pad(idx_ref, pos_ref, n):
            idx0 = plsc.load_gather(idx_ref, [zeros16])
            pos0 = plsc.load_gather(pos_ref, [zeros16])
            n8 = (n + 7) & -8
            pad_end = jnp.where(n8 < CH, jnp.int32(CH), n8)
            for kk in range(CH // L):
                slot = n + kk * L + lanes
                m = slot < pad_end
                plsc.store_scatter(idx_ref, [slot], idx0, mask=m)
                plsc.store_scatter(pos_ref, [slot], pos0, mask=m)
            return jnp.where(n > 0, pad_end, jnp.int32(0))

        ne_t = pad(tidx, tpos, nt)
        ne_i = pad(iidx, ipos, ni)

        nch_t = (ne_t + CH - 1) // CH
        nch_i = (ne_i + CH - 1) // CH
        total = nch_t + nch_i
        last_t = jnp.maximum(ne_t - CH, 0)
        last_i = jnp.maximum(ne_i - CH, 0)

        def start_gather(c, b):
            @pl.when(c < nch_t)
            def _():
                s = pl.multiple_of(jnp.minimum(c * CH, last_t), 8)
                pltpu.make_async_copy(
                    te_hbm.at[tidx.at[pl.ds(s, CH)]], b, sem_g).start()

            @pl.when(c >= nch_t)
            def _():
                s = pl.multiple_of(jnp.minimum((c - nch_t) * CH, last_i), 8)
                pltpu.make_async_copy(
                    pc_hbm.at[iidx.at[pl.ds(s, CH)]], b, sem_g).start()

        def wait_gather(b):
            pltpu.make_async_copy(
                te_hbm.at[tidx.at[pl.ds(0, CH)]], b, sem_g).wait()

        def start_scatter(c, b):
            @pl.when(c < nch_t)
            def _():
                s = pl.multiple_of(jnp.minimum(c * CH, last_t), 8)
                pltpu.make_async_copy(
                    b, out_hbm.at[tpos.at[pl.ds(s, CH)]], sem_s).start()

            @pl.when(c >= nch_t)
            def _():
                s = pl.multiple_of(jnp.minimum((c - nch_t) * CH, last_i), 8)
                pltpu.make_async_copy(
                    b, out_hbm.at[ipos.at[pl.ds(s, CH)]], sem_s).start()

        def wait_scatter(b):
            pltpu.make_async_copy(
                b, out_hbm.at[tpos.at[pl.ds(0, CH)]], sem_s).wait()

        for b in range(nb):
            @pl.when(b < total)
            def _(b=b):
                start_gather(b, bufs[b])

        def group(p, c):
            g0 = p * nb
            for b in range(nb):
                j = g0 + b

                @pl.when(j < total)
                def _(j=j, b=b):
                    wait_gather(bufs[b])
                    start_scatter(j, bufs[b])

                    @pl.when(j + nb < total)
                    def _():
                        wait_scatter(bufs[b])
                        start_gather(j + nb, bufs[b])
            return c

        lax.fori_loop(0, (total + nb - 1) // nb, group, 0)
        for b in range(nb):
            @pl.when(b < total)
            def _(b=b):
                wait_scatter(bufs[b])

    return k


def kernel(x, token_embedding, vqgan_codebook, vqgan_proj_W):
    pc = _project_codebook(vqgan_codebook, vqgan_proj_W)
    n_tokens = x.shape[0] * x.shape[1]
    out = _sc_lookup(n_tokens)(x.reshape(-1), token_embedding, pc)
    return out.reshape(x.shape + (EMBED,))


# speedup vs baseline: 1.0378x; 1.0027x over previous
import functools

import jax
import jax.numpy as jnp
from jax import lax
from jax.experimental import pallas as pl
from jax.experimental.pallas import tpu as pltpu
from jax.experimental.pallas import tpu_sc as plsc

EMBED = 1024
TEXT_END = 32000
IMG_OFFSET = 32000
L = 16
CH = 16
NB = 6


def _project_codebook(codebook, w):
    vq_vocab, vq_embed = codebook.shape
    bm = 512

    def body(cb_ref, w_ref, o_ref):
        o_ref[...] = lax.dot_general(
            cb_ref[...], w_ref[...],
            dimension_numbers=(((1,), (1,)), ((), ())),
            preferred_element_type=jnp.float32)

    return pl.pallas_call(
        body,
        grid=(vq_vocab // bm,),
        in_specs=[
            pl.BlockSpec((bm, vq_embed), lambda i: (i, 0)),
            pl.BlockSpec((EMBED, vq_embed), lambda i: (0, 0)),
        ],
        out_specs=pl.BlockSpec((bm, EMBED), lambda i: (i, 0)),
        out_shape=jax.ShapeDtypeStruct((vq_vocab, EMBED), jnp.float32),
    )(codebook, w)


@functools.cache
def _sc_lookup(n_tokens):
    info = plsc.get_sparse_core_info()
    nw = info.num_cores * info.num_subcores
    tpw = n_tokens // nw
    assert n_tokens % nw == 0 and tpw % L == 0
    nb = NB
    mesh = plsc.VectorSubcoreMesh(core_axis_name="c", subcore_axis_name="s")

    @functools.partial(
        pl.kernel,
        mesh=mesh,
        out_type=jax.ShapeDtypeStruct((n_tokens, EMBED), jnp.float32),
        compiler_params=pltpu.CompilerParams(needs_layout_passes=False),
        scratch_types=[
            pltpu.VMEM((tpw,), jnp.int32),
            pltpu.VMEM((tpw,), jnp.int32),
            pltpu.VMEM((tpw,), jnp.int32),
            pltpu.VMEM((tpw,), jnp.int32),
            pltpu.VMEM((tpw,), jnp.int32),
        ] + [pltpu.VMEM((CH, EMBED), jnp.float32)] * nb + [
            pltpu.SemaphoreType.DMA,
            pltpu.SemaphoreType.DMA,
        ],
    )
    def k(x_hbm, te_hbm, pc_hbm, out_hbm,
          x_v, tidx, tpos, iidx, ipos, *rest):
        bufs = rest[:nb]
        sem_g, sem_s = rest[nb], rest[nb + 1]
        wid = lax.axis_index("s") * info.num_cores + lax.axis_index("c")
        base = wid * tpw
        pltpu.sync_copy(x_hbm.at[pl.ds(base, tpw)], x_v)

        lanes = lax.iota(jnp.int32, L)

        def compact(j, carry):
            nt, ni = carry
            xv = x_v[pl.ds(j * L, L)]
            m_text = xv < TEXT_END
            m_img = jnp.logical_not(m_text)
            mt32 = m_text.astype(jnp.int32)
            incl = plsc.cumsum(mt32)
            excl = incl - mt32
            pos = base + j * L + lanes
            slot_t = nt + excl
            slot_i = ni + (lanes - excl)
            plsc.store_scatter(tidx, [slot_t], xv, mask=m_text)
            plsc.store_scatter(tpos, [slot_t], pos, mask=m_text)
            plsc.store_scatter(iidx, [slot_i], xv - IMG_OFFSET, mask=m_img)
            plsc.store_scatter(ipos, [slot_i], pos, mask=m_img)
            cnt = incl[L - 1]
            return nt + cnt, ni + (L - cnt)

        nt, ni = lax.fori_loop(0, tpw // L, compact,
                               (jnp.int32(0), jnp.int32(0)))

        zeros16 = jnp.zeros((L,), jnp.int32)

        def pad(idx_ref, pos_ref, n):
            idx0 = plsc.load_gather(idx_ref, [zeros16])
            pos0 = plsc.load_gather(pos_ref, [zeros16])
            n8 = (n + 7) & -8
            pad_end = jnp.where(n8 < CH, jnp.int32(CH), n8)
            for kk in range(CH // L):
                slot = n + kk * L + lanes
                m = slot < pad_end
                plsc.store_scatter(idx_ref, [slot], idx0, mask=m)
                plsc.store_scatter(pos_ref, [slot], pos0, mask=m)
            return jnp.where(n > 0, pad_end, jnp.int32(0))

        ne_t = pad(tidx, tpos, nt)
        ne_i = pad(iidx, ipos, ni)

        nch_t = (ne_t + CH - 1) // CH
        nch_i = (ne_i + CH - 1) // CH
        total = nch_t + nch_i
        last_t = jnp.maximum(ne_t - CH, 0)
        last_i = jnp.maximum(ne_i - CH, 0)

        def start_gather(c, b):
            @pl.when(c < nch_t)
            def _():
                s = pl.multiple_of(jnp.minimum(c * CH, last_t), 8)
                pltpu.make_async_copy(
                    te_hbm.at[tidx.at[pl.ds(s, CH)]], b, sem_g).start()

            @pl.when(c >= nch_t)
            def _():
                s = pl.multiple_of(jnp.minimum((c - nch_t) * CH, last_i), 8)
                pltpu.make_async_copy(
                    pc_hbm.at[iidx.at[pl.ds(s, CH)]], b, sem_g).start()

        def wait_gather(b):
            pltpu.make_async_copy(
                te_hbm.at[tidx.at[pl.ds(0, CH)]], b, sem_g).wait()

        def start_scatter(c, b):
            @pl.when(c < nch_t)
            def _():
                s = pl.multiple_of(jnp.minimum(c * CH, last_t), 8)
                pltpu.make_async_copy(
                    b, out_hbm.at[tpos.at[pl.ds(s, CH)]], sem_s).start()

            @pl.when(c >= nch_t)
            def _():
                s = pl.multiple_of(jnp.minimum((c - nch_t) * CH, last_i), 8)
                pltpu.make_async_copy(
                    b, out_hbm.at[ipos.at[pl.ds(s, CH)]], sem_s).start()

        def wait_scatter(b):
            pltpu.make_async_copy(
                b, out_hbm.at[tpos.at[pl.ds(0, CH)]], sem_s).wait()

        for b in range(nb):
            @pl.when(b < total)
            def _(b=b):
                start_gather(b, bufs[b])

        def group(p, c):
            g0 = p * nb
            for b in range(nb):
                j = g0 + b

                @pl.when(j < total)
                def _(j=j, b=b):
                    wait_gather(bufs[b])
                    start_scatter(j, bufs[b])

                    @pl.when(j + nb < total)
                    def _():
                        wait_scatter(bufs[b])
                        start_gather(j + nb, bufs[b])
            return c

        lax.fori_loop(0, (total + nb - 1) // nb, group, 0)
        for b in range(nb):
            @pl.when(b < total)
            def _(b=b):
                wait_scatter(bufs[b])

    return k


def kernel(x, token_embedding, vqgan_codebook, vqgan_proj_W):
    pc = _project_codebook(vqgan_codebook, vqgan_proj_W)
    n_tokens = x.shape[0] * x.shape[1]
    out = _sc_lookup(n_tokens)(x.reshape(-1), token_embedding, pc)
    return out.reshape(x.shape + (EMBED,))


# speedup vs baseline: 1.0399x; 1.0020x over previous
import functools

import jax
import jax.numpy as jnp
from jax import lax
from jax.experimental import pallas as pl
from jax.experimental.pallas import tpu as pltpu
from jax.experimental.pallas import tpu_sc as plsc

EMBED = 1024
TEXT_END = 32000
IMG_OFFSET = 32000
L = 16
CH = 16
NB = 7


def _project_codebook(codebook, w):
    vq_vocab, vq_embed = codebook.shape
    bm = 512

    def body(cb_ref, w_ref, o_ref):
        o_ref[...] = lax.dot_general(
            cb_ref[...], w_ref[...],
            dimension_numbers=(((1,), (1,)), ((), ())),
            preferred_element_type=jnp.float32)

    return pl.pallas_call(
        body,
        grid=(vq_vocab // bm,),
        in_specs=[
            pl.BlockSpec((bm, vq_embed), lambda i: (i, 0)),
            pl.BlockSpec((EMBED, vq_embed), lambda i: (0, 0)),
        ],
        out_specs=pl.BlockSpec((bm, EMBED), lambda i: (i, 0)),
        out_shape=jax.ShapeDtypeStruct((vq_vocab, EMBED), jnp.float32),
    )(codebook, w)


@functools.cache
def _sc_lookup(n_tokens):
    info = plsc.get_sparse_core_info()
    nw = info.num_cores * info.num_subcores
    tpw = n_tokens // nw
    assert n_tokens % nw == 0 and tpw % L == 0
    nb = NB
    mesh = plsc.VectorSubcoreMesh(core_axis_name="c", subcore_axis_name="s")

    @functools.partial(
        pl.kernel,
        mesh=mesh,
        out_type=jax.ShapeDtypeStruct((n_tokens, EMBED), jnp.float32),
        compiler_params=pltpu.CompilerParams(needs_layout_passes=False),
        scratch_types=[
            pltpu.VMEM((tpw,), jnp.int32),
            pltpu.VMEM((tpw,), jnp.int32),
            pltpu.VMEM((tpw,), jnp.int32),
            pltpu.VMEM((tpw,), jnp.int32),
            pltpu.VMEM((tpw,), jnp.int32),
        ] + [pltpu.VMEM((CH, EMBED), jnp.float32)] * nb + [
            pltpu.SemaphoreType.DMA,
            pltpu.SemaphoreType.DMA,
        ],
    )
    def k(x_hbm, te_hbm, pc_hbm, out_hbm,
          x_v, tidx, tpos, iidx, ipos, *rest):
        bufs = rest[:nb]
        sem_g, sem_s = rest[nb], rest[nb + 1]
        wid = lax.axis_index("s") * info.num_cores + lax.axis_index("c")
        base = wid * tpw
        pltpu.sync_copy(x_hbm.at[pl.ds(base, tpw)], x_v)

        lanes = lax.iota(jnp.int32, L)

        def compact(j, carry):
            nt, ni = carry
            xv = x_v[pl.ds(j * L, L)]
            m_text = xv < TEXT_END
            m_img = jnp.logical_not(m_text)
            mt32 = m_text.astype(jnp.int32)
            incl = plsc.cumsum(mt32)
            excl = incl - mt32
            pos = base + j * L + lanes
            slot_t = nt + excl
            slot_i = ni + (lanes - excl)
            plsc.store_scatter(tidx, [slot_t], xv, mask=m_text)
            plsc.store_scatter(tpos, [slot_t], pos, mask=m_text)
            plsc.store_scatter(iidx, [slot_i], xv - IMG_OFFSET, mask=m_img)
            plsc.store_scatter(ipos, [slot_i], pos, mask=m_img)
            cnt = incl[L - 1]
            return nt + cnt, ni + (L - cnt)

        nt, ni = lax.fori_loop(0, tpw // L, compact,
                               (jnp.int32(0), jnp.int32(0)))

        zeros16 = jnp.zeros((L,), jnp.int32)

        def pad(idx_ref, pos_ref, n):
            idx0 = plsc.load_gather(idx_ref, [zeros16])
            pos0 = plsc.load_gather(pos_ref, [zeros16])
            n8 = (n + 7) & -8
            pad_end = jnp.where(n8 < CH, jnp.int32(CH), n8)
            for kk in range(CH // L):
                slot = n + kk * L + lanes
                m = slot < pad_end
                plsc.store_scatter(idx_ref, [slot], idx0, mask=m)
                plsc.store_scatter(pos_ref, [slot], pos0, mask=m)
            return jnp.where(n > 0, pad_end, jnp.int32(0))

        ne_t = pad(tidx, tpos, nt)
        ne_i = pad(iidx, ipos, ni)

        nch_t = (ne_t + CH - 1) // CH
        nch_i = (ne_i + CH - 1) // CH
        total = nch_t + nch_i
        last_t = jnp.maximum(ne_t - CH, 0)
        last_i = jnp.maximum(ne_i - CH, 0)

        def start_gather(c, b):
            @pl.when(c < nch_t)
            def _():
                s = pl.multiple_of(jnp.minimum(c * CH, last_t), 8)
                pltpu.make_async_copy(
                    te_hbm.at[tidx.at[pl.ds(s, CH)]], b, sem_g).start()

            @pl.when(c >= nch_t)
            def _():
                s = pl.multiple_of(jnp.minimum((c - nch_t) * CH, last_i), 8)
                pltpu.make_async_copy(
                    pc_hbm.at[iidx.at[pl.ds(s, CH)]], b, sem_g).start()

        def wait_gather(b):
            pltpu.make_async_copy(
                te_hbm.at[tidx.at[pl.ds(0, CH)]], b, sem_g).wait()

        def start_scatter(c, b):
            @pl.when(c < nch_t)
            def _():
                s = pl.multiple_of(jnp.minimum(c * CH, last_t), 8)
                pltpu.make_async_copy(
                    b, out_hbm.at[tpos.at[pl.ds(s, CH)]], sem_s).start()

            @pl.when(c >= nch_t)
            def _():
                s = pl.multiple_of(jnp.minimum((c - nch_t) * CH, last_i), 8)
                pltpu.make_async_copy(
                    b, out_hbm.at[ipos.at[pl.ds(s, CH)]], sem_s).start()

        def wait_scatter(b):
            pltpu.make_async_copy(
                b, out_hbm.at[tpos.at[pl.ds(0, CH)]], sem_s).wait()

        for b in range(nb):
            @pl.when(b < total)
            def _(b=b):
                start_gather(b, bufs[b])

        def group(p, c):
            g0 = p * nb
            for b in range(nb):
                j = g0 + b

                @pl.when(j < total)
                def _(j=j, b=b):
                    wait_gather(bufs[b])
                    start_scatter(j, bufs[b])

                    @pl.when(j + nb < total)
                    def _():
                        wait_scatter(bufs[b])
                        start_gather(j + nb, bufs[b])
            return c

        lax.fori_loop(0, (total + nb - 1) // nb, group, 0)
        for b in range(nb):
            @pl.when(b < total)
            def _(b=b):
                wait_scatter(bufs[b])

    return k


def kernel(x, token_embedding, vqgan_codebook, vqgan_proj_W):
    pc = _project_codebook(vqgan_codebook, vqgan_proj_W)
    n_tokens = x.shape[0] * x.shape[1]
    out = _sc_lookup(n_tokens)(x.reshape(-1), token_embedding, pc)
    return out.reshape(x.shape + (EMBED,))
